# baseline (speedup 1.0000x reference)
import functools

import jax
import jax.numpy as jnp
from jax import lax
from jax.experimental import pallas as pl
from jax.experimental.pallas import tpu as pltpu
from jax.experimental.pallas import tpu_sc as plsc

_IDX_CHUNK = 128
_NPOW = 4


def _make_sc_gather_powers(n_turns, n_tok, vocab):
    info = plsc.get_sparse_core_info()
    nw = info.num_cores * info.num_subcores
    npw = _NPOW * n_turns
    tok_per_w = n_tok // nw
    lanes = info.num_lanes
    wpw = tok_per_w * n_turns
    n_chunks = wpw // _IDX_CHUNK
    pairs = tok_per_w // (lanes // n_turns)
    mesh = plsc.VectorSubcoreMesh(core_axis_name="c", subcore_axis_name="s")

    @functools.partial(
        pl.kernel,
        mesh=mesh,
        compiler_params=pltpu.CompilerParams(
            use_tc_tiling_on_sc=False, needs_layout_passes=False
        ),
        out_type=jax.ShapeDtypeStruct((n_tok, npw), jnp.float32),
        scratch_types=[
            pltpu.VMEM((tok_per_w,), jnp.int32),
            pltpu.VMEM((wpw,), jnp.int32),
            pltpu.VMEM((wpw,), jnp.float32),
            pltpu.VMEM((tok_per_w, npw), jnp.float32),
            pltpu.SemaphoreType.DMA,
        ],
    )
    def gather_kernel(turns_hbm, ids_hbm, out_hbm, ids_v, widx_v, rows_v, pow_v, sem):
        wid = lax.axis_index("s") * info.num_cores + lax.axis_index("c")
        base = wid * tok_per_w
        pltpu.sync_copy(ids_hbm.at[pl.ds(base, tok_per_w)], ids_v)

        for t in range(n_turns):
            for i in range(tok_per_w // lanes):
                widx_v[pl.ds(t * tok_per_w + i * lanes, lanes)] = (
                    ids_v[pl.ds(i * lanes, lanes)] + t * vocab
                )
        copies = [
            pltpu.async_copy(
                turns_hbm.at[widx_v.at[pl.ds(c * _IDX_CHUNK, _IDX_CHUNK)]],
                rows_v.at[pl.ds(c * _IDX_CHUNK, _IDX_CHUNK)],
                sem,
            )
            for c in range(n_chunks)
        ]
        for cp in copies:
            cp.wait()

        lane = lax.broadcasted_iota(jnp.int32, (lanes,), 0)
        half = lane >> 3
        within = lane & (n_turns - 1)
        cols = [within + d * n_turns for d in range(_NPOW)]

        def pow_body(k, _):
            row = 2 * k + half
            x = plsc.load_gather(rows_v, [within * tok_per_w + row])
            x2 = x * x
            x3 = x2 * x
            x4 = x2 * x2
            for d, v in enumerate((x, x2, x3, x4)):
                plsc.store_scatter(pow_v, [row, cols[d]], v)
            return 0

        lax.fori_loop(0, pairs, pow_body, 0, unroll=4)
        pltpu.sync_copy(pow_v, out_hbm.at[pl.ds(base, tok_per_w)])

    return gather_kernel


def _tc_body(p_ref, pc_ref, pos_ref, g_ref, b_ref, o_ref):
    p = p_ref[...]
    pc = pc_ref[...]
    t = pc.shape[0] - p.shape[-1]
    c0 = jnp.sum(pc[0:t], axis=0, keepdims=True)
    emb = jnp.dot(p, pc[t:], preferred_element_type=jnp.float32)
    emb = emb + c0 + pos_ref[...]
    mean = jnp.mean(emb, axis=-1, keepdims=True)
    cen = emb - mean
    var = jnp.mean(cen * cen, axis=-1, keepdims=True)
    o_ref[...] = cen * lax.rsqrt(var + 1e-12) * g_ref[...] + b_ref[...]


_BS = 1024


def _tc_dense(p2d, pc, pos_table, gamma, beta, b, s):
    d = pos_table.shape[-1]
    nsb = s // _BS
    npw = p2d.shape[-1]
    return pl.pallas_call(
        _tc_body,
        grid=(nsb, b),
        in_specs=[
            pl.BlockSpec((_BS, npw), lambda j, i: (i * nsb + j, 0)),
            pl.BlockSpec((pc.shape[0], d), lambda j, i: (0, 0)),
            pl.BlockSpec((_BS, d), lambda j, i: (j, 0)),
            pl.BlockSpec((1, d), lambda j, i: (0, 0)),
            pl.BlockSpec((1, d), lambda j, i: (0, 0)),
        ],
        out_specs=pl.BlockSpec((_BS, d), lambda j, i: (i * nsb + j, 0)),
        out_shape=jax.ShapeDtypeStruct((b * s, d), jnp.float32),
        compiler_params=pltpu.CompilerParams(
            dimension_semantics=("parallel", "parallel"),
        ),
    )(p2d, pc, pos_table, gamma, beta).reshape(b, s, d)


def kernel(input_ids, turns, poly_coeffs, pos_table, ln_gamma, ln_beta):
    b, s = input_ids.shape
    vocab, n_turns = turns.shape
    n_tok = b * s
    ids = input_ids.astype(jnp.int32).reshape(n_tok)
    turns_flat = turns.T.reshape(-1)
    p2d = _make_sc_gather_powers(n_turns, n_tok, vocab)(turns_flat, ids)
    pc = jnp.transpose(poly_coeffs, (1, 0, 2)).reshape(-1, poly_coeffs.shape[-1])
    return _tc_dense(
        p2d,
        pc,
        pos_table,
        ln_gamma.reshape(1, -1),
        ln_beta.reshape(1, -1),
        b,
        s,
    )


# speedup vs baseline: 1.0905x; 1.0905x over previous
import functools

import jax
import jax.numpy as jnp
from jax import lax
from jax.experimental import pallas as pl
from jax.experimental.pallas import tpu as pltpu
from jax.experimental.pallas import tpu_sc as plsc

_IDX_CHUNK = 128
_NPOW = 4


def _make_sc_gather_powers(n_turns, n_tok, vocab):
    info = plsc.get_sparse_core_info()
    nw = info.num_cores * info.num_subcores
    npw = _NPOW * n_turns
    tok_per_w = n_tok // nw
    lanes = info.num_lanes
    wpw = tok_per_w * n_turns
    n_chunks = wpw // _IDX_CHUNK
    pairs = tok_per_w // (lanes // n_turns)
    mesh = plsc.VectorSubcoreMesh(core_axis_name="c", subcore_axis_name="s")

    @functools.partial(
        pl.kernel,
        mesh=mesh,
        compiler_params=pltpu.CompilerParams(
            use_tc_tiling_on_sc=False, needs_layout_passes=False
        ),
        out_type=jax.ShapeDtypeStruct((n_tok // 4, 4 * npw), jnp.float32),
        scratch_types=[
            pltpu.VMEM((tok_per_w,), jnp.int32),
            pltpu.VMEM((wpw,), jnp.int32),
            pltpu.VMEM((wpw,), jnp.float32),
            pltpu.VMEM((tok_per_w, npw), jnp.float32),
            pltpu.SemaphoreType.DMA,
        ],
    )
    def gather_kernel(turns_hbm, ids_hbm, out_hbm, ids_v, widx_v, rows_v, pow_v, sem):
        wid = lax.axis_index("s") * info.num_cores + lax.axis_index("c")
        base = wid * tok_per_w
        pltpu.sync_copy(ids_hbm.at[pl.ds(base, tok_per_w)], ids_v)

        for t in range(n_turns):
            for i in range(tok_per_w // lanes):
                widx_v[pl.ds(t * tok_per_w + i * lanes, lanes)] = (
                    ids_v[pl.ds(i * lanes, lanes)] + t * vocab
                )
        copies = [
            pltpu.async_copy(
                turns_hbm.at[widx_v.at[pl.ds(c * _IDX_CHUNK, _IDX_CHUNK)]],
                rows_v.at[pl.ds(c * _IDX_CHUNK, _IDX_CHUNK)],
                sem,
            )
            for c in range(n_chunks)
        ]
        for cp in copies:
            cp.wait()

        lane = lax.broadcasted_iota(jnp.int32, (lanes,), 0)
        half = lane >> 3
        within = lane & (n_turns - 1)
        cols = [within + d * n_turns for d in range(_NPOW)]

        def pow_body(k, _):
            row = 2 * k + half
            x = plsc.load_gather(rows_v, [within * tok_per_w + row])
            x2 = x * x
            x3 = x2 * x
            x4 = x2 * x2
            for d, v in enumerate((x, x2, x3, x4)):
                plsc.store_scatter(pow_v, [row, cols[d]], v)
            return 0

        lax.fori_loop(0, pairs, pow_body, 0, unroll=4)
        grp = wid // (nw // 4)
        rbase = (wid % (nw // 4)) * tok_per_w
        pltpu.sync_copy(
            pow_v,
            out_hbm.at[pl.ds(rbase, tok_per_w), pl.ds(grp * npw, npw)],
        )

    return gather_kernel


def _tc_body(p_ref, pc4_ref, pc_ref, pos_ref, g_ref, b_ref, o_ref):
    p = p_ref[...]
    pc4 = pc4_ref[0]
    pc = pc_ref[...]
    c0 = jnp.sum(pc[0:8], axis=0, keepdims=True)
    emb = jnp.dot(p, pc4, preferred_element_type=jnp.float32)
    emb = emb + c0 + pos_ref[...]
    mean = jnp.mean(emb, axis=-1, keepdims=True)
    cen = emb - mean
    var = jnp.mean(cen * cen, axis=-1, keepdims=True)
    o_ref[...] = cen * lax.rsqrt(var + 1e-12) * g_ref[...] + b_ref[...]


_BS = 1024


def _tc_dense(p2d, pc4, pc, pos_table, gamma, beta, b, s):
    d = pos_table.shape[-1]
    nsb = s // _BS
    npw4 = p2d.shape[-1]
    return pl.pallas_call(
        _tc_body,
        grid=(nsb, b),
        in_specs=[
            pl.BlockSpec((_BS, npw4), lambda j, i: ((i * nsb + j) % 4, 0)),
            pl.BlockSpec((1, npw4, d), lambda j, i: ((i * nsb + j) // 4, 0, 0)),
            pl.BlockSpec((pc.shape[0], d), lambda j, i: (0, 0)),
            pl.BlockSpec((_BS, d), lambda j, i: (j, 0)),
            pl.BlockSpec((1, d), lambda j, i: (0, 0)),
            pl.BlockSpec((1, d), lambda j, i: (0, 0)),
        ],
        out_specs=pl.BlockSpec((_BS, d), lambda j, i: (i * nsb + j, 0)),
        out_shape=jax.ShapeDtypeStruct((b * s, d), jnp.float32),
        compiler_params=pltpu.CompilerParams(
            dimension_semantics=("parallel", "parallel"),
        ),
    )(p2d, pc4, pc, pos_table, gamma, beta).reshape(b, s, d)


def kernel(input_ids, turns, poly_coeffs, pos_table, ln_gamma, ln_beta):
    b, s = input_ids.shape
    vocab, n_turns = turns.shape
    n_tok = b * s
    ids = input_ids.astype(jnp.int32).reshape(n_tok)
    turns_flat = turns.T.reshape(-1)
    p2d = _make_sc_gather_powers(n_turns, n_tok, vocab)(turns_flat, ids)
    pc = jnp.transpose(poly_coeffs, (1, 0, 2)).reshape(-1, poly_coeffs.shape[-1])
    npw = _NPOW * n_turns
    pck = pc[n_turns:]
    pc4 = jnp.stack(
        [
            jnp.pad(pck, ((q * npw, (3 - q) * npw), (0, 0)))
            for q in range(4)
        ]
    )
    return _tc_dense(
        p2d,
        pc4,
        pc,
        pos_table,
        ln_gamma.reshape(1, -1),
        ln_beta.reshape(1, -1),
        b,
        s,
    )
